# baseline (speedup 1.0000x reference)
import jax
import jax.numpy as jnp
from jax.experimental import pallas as pl
from jax.experimental.pallas import tpu as pltpu

Q_MIN = 0.1
SUPRESSION = 0.1
COND_WEIGHT = 1.0

_B, _NOBJ, _H, _W = 8, 8, 128, 128
_N = _H * _W
_IPS = 2


def _atanh(v):
    return jnp.log1p(2.0 * v / (1.0 - v)) / 2.0


def _image_loss(xv, beta, m3, inp, tgt, lin, e0, e1):
    x0 = jax.lax.dot_general(xv, e0, (((1,), (0,)), ((), ())),
                             precision=jax.lax.Precision.HIGHEST)
    x1 = jax.lax.dot_general(xv, e1, (((1,), (0,)), ((), ())),
                             precision=jax.lax.Precision.HIGHEST)
    q = _atanh(beta) ** 2 + Q_MIN
    loss_elem = (inp - tgt) ** 2

    msum = jnp.sum(m3, axis=0)
    noise = (msum < 1.0).astype(jnp.float32)
    sel = 1.0 - noise
    qq = q - Q_MIN

    selq = sel * qq
    temp_div = jnp.sum(selq)
    temp_loss = jnp.sum(selq * loss_elem)
    gen = jnp.where(temp_div == 0.0, 0.0,
                    temp_loss / jnp.where(temp_div == 0.0, 1.0, temp_div))

    n_b = jnp.sum(noise)
    bg_noise = SUPRESSION * jnp.sum(noise * beta) / n_b
    ba3 = jnp.max(beta[None] * m3, axis=(1, 2))

    qm3 = q[None] * m3
    qa3 = jnp.max(qm3, axis=(1, 2), keepdims=True)
    idx3 = jnp.min(jnp.where(qm3 == qa3, lin[None], jnp.float32(_N)),
                   axis=(1, 2), keepdims=True)
    oh3 = (lin[None] == idx3).astype(jnp.float32)
    a03 = jnp.sum(oh3 * x0[None], axis=(1, 2), keepdims=True)
    a13 = jnp.sum(oh3 * x1[None], axis=(1, 2), keepdims=True)

    d0 = x0[None] - a03
    d1 = x1[None] - a13
    n2 = d0 * d0 + d1 * d1
    xn = jnp.sqrt(n2)
    rep = jnp.maximum(1.0 - xn, 0.0)
    blend = rep + m3 * (n2 - rep)
    temp = jnp.sum(qa3 * blend, axis=0)
    pot = jnp.sum(q * temp) * (1.0 / _N)

    bg = (1.0 - jnp.sum(ba3) * (1.0 / _NOBJ)) + bg_noise
    return gen + COND_WEIGHT * (bg + pot)


def _loss_kernel(x_ref, beta_ref, m_ref, inp_ref, tgt_ref, out_ref):
    lin = (jax.lax.broadcasted_iota(jnp.int32, (_H, _W), 0) * _W
           + jax.lax.broadcasted_iota(jnp.int32, (_H, _W), 1)
           ).astype(jnp.float32)
    k_iota = jax.lax.broadcasted_iota(jnp.int32, (2 * _W, _W), 0)
    j_iota = jax.lax.broadcasted_iota(jnp.int32, (2 * _W, _W), 1)
    e0 = (k_iota == 2 * j_iota).astype(jnp.float32)
    e1 = (k_iota == 2 * j_iota + 1).astype(jnp.float32)

    acc = 0.0
    for j in range(_IPS):
        acc += _image_loss(x_ref[j], beta_ref[j], m_ref[j],
                           inp_ref[j, 0], tgt_ref[j, 0], lin, e0, e1)

    pid = pl.program_id(0)

    @pl.when(pid == 0)
    def _init():
        out_ref[...] = jnp.zeros((1, 1), jnp.float32)

    out_ref[...] += jnp.full((1, 1), acc * (1.0 / _B), jnp.float32)


def kernel(x, beta, matrix, input, target):
    B, n_objects, H, W = matrix.shape
    xr = x.reshape(B, H, 2 * W)

    out = pl.pallas_call(
        _loss_kernel,
        grid=(B // _IPS,),
        in_specs=[
            pl.BlockSpec((_IPS, H, 2 * W), lambda b: (b, 0, 0)),
            pl.BlockSpec((_IPS, H, W), lambda b: (b, 0, 0)),
            pl.BlockSpec((_IPS, n_objects, H, W), lambda b: (b, 0, 0, 0)),
            pl.BlockSpec((_IPS, 1, H, W), lambda b: (b, 0, 0, 0)),
            pl.BlockSpec((_IPS, 1, H, W), lambda b: (b, 0, 0, 0)),
        ],
        out_specs=pl.BlockSpec((1, 1), lambda b: (0, 0)),
        out_shape=jax.ShapeDtypeStruct((1, 1), jnp.float32),
        compiler_params=pltpu.CompilerParams(
            dimension_semantics=("arbitrary",)),
    )(xr, beta, matrix, input, target)
    return out.reshape(())


# speedup vs baseline: 1.0754x; 1.0754x over previous
import jax
import jax.numpy as jnp
from jax.experimental import pallas as pl
from jax.experimental.pallas import tpu as pltpu

Q_MIN = 0.1
SUPRESSION = 0.1
COND_WEIGHT = 1.0

_B, _NOBJ, _H, _W = 8, 8, 128, 128
_N = _H * _W
_IPS = 2


def _atanh(v):
    return jnp.log1p(2.0 * v / (1.0 - v)) / 2.0


def _image_loss(x0, x1, beta, m3, inp, tgt, lin):
    q = _atanh(beta) ** 2 + Q_MIN
    loss_elem = (inp - tgt) ** 2

    msum = jnp.sum(m3, axis=0)
    noise = (msum < 1.0).astype(jnp.float32)
    sel = 1.0 - noise
    qq = q - Q_MIN

    selq = sel * qq
    temp_div = jnp.sum(selq)
    temp_loss = jnp.sum(selq * loss_elem)
    gen = jnp.where(temp_div == 0.0, 0.0,
                    temp_loss / jnp.where(temp_div == 0.0, 1.0, temp_div))

    n_b = jnp.sum(noise)
    bg_noise = SUPRESSION * jnp.sum(noise * beta) / n_b
    ba3 = jnp.max(beta[None] * m3, axis=(1, 2))

    qm3 = q[None] * m3
    qa3 = jnp.max(qm3, axis=(1, 2), keepdims=True)
    idx3 = jnp.min(jnp.where(qm3 == qa3, lin[None], jnp.float32(_N)),
                   axis=(1, 2), keepdims=True)
    oh3 = (lin[None] == idx3).astype(jnp.float32)
    a03 = jnp.sum(oh3 * x0[None], axis=(1, 2), keepdims=True)
    a13 = jnp.sum(oh3 * x1[None], axis=(1, 2), keepdims=True)

    d0 = x0[None] - a03
    d1 = x1[None] - a13
    n2 = d0 * d0 + d1 * d1
    xn = jnp.sqrt(n2)
    rep = jnp.maximum(1.0 - xn, 0.0)
    blend = rep + m3 * (n2 - rep)
    temp = jnp.sum(qa3 * blend, axis=0)
    pot = jnp.sum(q * temp) * (1.0 / _N)

    bg = (1.0 - jnp.sum(ba3) * (1.0 / _NOBJ)) + bg_noise
    return gen + COND_WEIGHT * (bg + pot)


def _loss_kernel(x0_ref, x1_ref, beta_ref, m_ref, inp_ref, tgt_ref, out_ref):
    lin = (jax.lax.broadcasted_iota(jnp.int32, (_H, _W), 0) * _W
           + jax.lax.broadcasted_iota(jnp.int32, (_H, _W), 1)
           ).astype(jnp.float32)

    acc = 0.0
    for j in range(_IPS):
        acc += _image_loss(x0_ref[j], x1_ref[j], beta_ref[j], m_ref[j],
                           inp_ref[j, 0], tgt_ref[j, 0], lin)

    pid = pl.program_id(0)

    @pl.when(pid == 0)
    def _init():
        out_ref[...] = jnp.zeros((1, 1), jnp.float32)

    out_ref[...] += jnp.full((1, 1), acc * (1.0 / _B), jnp.float32)


def kernel(x, beta, matrix, input, target):
    B, n_objects, H, W = matrix.shape
    x0 = x[..., 0]
    x1 = x[..., 1]

    out = pl.pallas_call(
        _loss_kernel,
        grid=(B // _IPS,),
        in_specs=[
            pl.BlockSpec((_IPS, H, W), lambda b: (b, 0, 0)),
            pl.BlockSpec((_IPS, H, W), lambda b: (b, 0, 0)),
            pl.BlockSpec((_IPS, H, W), lambda b: (b, 0, 0)),
            pl.BlockSpec((_IPS, n_objects, H, W), lambda b: (b, 0, 0, 0)),
            pl.BlockSpec((_IPS, 1, H, W), lambda b: (b, 0, 0, 0)),
            pl.BlockSpec((_IPS, 1, H, W), lambda b: (b, 0, 0, 0)),
        ],
        out_specs=pl.BlockSpec((1, 1), lambda b: (0, 0)),
        out_shape=jax.ShapeDtypeStruct((1, 1), jnp.float32),
        compiler_params=pltpu.CompilerParams(
            dimension_semantics=("arbitrary",)),
    )(x0, x1, beta, matrix, input, target)
    return out.reshape(())


# speedup vs baseline: 1.2455x; 1.1582x over previous
import jax
import jax.numpy as jnp
from jax.experimental import pallas as pl
from jax.experimental.pallas import tpu as pltpu

Q_MIN = 0.1
SUPRESSION = 0.1
COND_WEIGHT = 1.0

_B, _NOBJ, _H, _W = 8, 8, 128, 128
_N = _H * _W
_IPS = 2


def _atanh(v):
    return jnp.log1p(2.0 * v / (1.0 - v)) / 2.0


def _image_loss(x0, x1, beta, m3, inp, tgt, lin):
    q = _atanh(beta) ** 2 + Q_MIN
    loss_elem = (inp - tgt) ** 2

    msum = jnp.sum(m3, axis=0)
    noise = (msum < 1.0).astype(jnp.float32)
    sel = 1.0 - noise
    qq = q - Q_MIN

    selq = sel * qq
    temp_div = jnp.sum(selq)
    temp_loss = jnp.sum(selq * loss_elem)
    gen = jnp.where(temp_div == 0.0, 0.0,
                    temp_loss / jnp.where(temp_div == 0.0, 1.0, temp_div))

    n_b = jnp.sum(noise)
    bg_noise = SUPRESSION * jnp.sum(noise * beta) / n_b
    ba3 = jnp.max(beta[None] * m3, axis=(1, 2))

    qm3 = q[None] * m3
    qa3 = jnp.max(qm3, axis=(1, 2), keepdims=True)
    idx3 = jnp.min(jnp.where(qm3 == qa3, lin[None], jnp.float32(_N)),
                   axis=(1, 2), keepdims=True)
    oh3 = (lin[None] == idx3).astype(jnp.float32)
    a03 = jnp.sum(oh3 * x0[None], axis=(1, 2), keepdims=True)
    a13 = jnp.sum(oh3 * x1[None], axis=(1, 2), keepdims=True)

    bf = jnp.bfloat16
    d0 = x0[None].astype(bf) - a03.astype(bf)
    d1 = x1[None].astype(bf) - a13.astype(bf)
    n2 = d0 * d0 + d1 * d1
    xn = jnp.sqrt(n2)
    rep = jnp.maximum(jnp.asarray(1.0, bf) - xn, jnp.asarray(0.0, bf))
    blend = rep + m3.astype(bf) * (n2 - rep)
    temp = jnp.sum(qa3.astype(bf) * blend, axis=0)
    pot = jnp.sum(q * temp.astype(jnp.float32)) * (1.0 / _N)

    bg = (1.0 - jnp.sum(ba3) * (1.0 / _NOBJ)) + bg_noise
    return gen + COND_WEIGHT * (bg + pot)


def _loss_kernel(x0_ref, x1_ref, beta_ref, m_ref, inp_ref, tgt_ref, out_ref):
    lin = (jax.lax.broadcasted_iota(jnp.int32, (_H, _W), 0) * _W
           + jax.lax.broadcasted_iota(jnp.int32, (_H, _W), 1)
           ).astype(jnp.float32)

    acc = 0.0
    for j in range(_IPS):
        acc += _image_loss(x0_ref[j], x1_ref[j], beta_ref[j], m_ref[j],
                           inp_ref[j, 0], tgt_ref[j, 0], lin)

    pid = pl.program_id(0)

    @pl.when(pid == 0)
    def _init():
        out_ref[...] = jnp.zeros((1, 1), jnp.float32)

    out_ref[...] += jnp.full((1, 1), acc * (1.0 / _B), jnp.float32)


def kernel(x, beta, matrix, input, target):
    B, n_objects, H, W = matrix.shape
    x0 = x[..., 0]
    x1 = x[..., 1]

    out = pl.pallas_call(
        _loss_kernel,
        grid=(B // _IPS,),
        in_specs=[
            pl.BlockSpec((_IPS, H, W), lambda b: (b, 0, 0)),
            pl.BlockSpec((_IPS, H, W), lambda b: (b, 0, 0)),
            pl.BlockSpec((_IPS, H, W), lambda b: (b, 0, 0)),
            pl.BlockSpec((_IPS, n_objects, H, W), lambda b: (b, 0, 0, 0)),
            pl.BlockSpec((_IPS, 1, H, W), lambda b: (b, 0, 0, 0)),
            pl.BlockSpec((_IPS, 1, H, W), lambda b: (b, 0, 0, 0)),
        ],
        out_specs=pl.BlockSpec((1, 1), lambda b: (0, 0)),
        out_shape=jax.ShapeDtypeStruct((1, 1), jnp.float32),
        compiler_params=pltpu.CompilerParams(
            dimension_semantics=("arbitrary",)),
    )(x0, x1, beta, matrix, input, target)
    return out.reshape(())
